# Initial kernel scaffold; baseline (speedup 1.0000x reference)
import jax
import jax.numpy as jnp
from jax.experimental import pallas as pl


def kernel(pred, gt, weights):
    raise NotImplementedError("write your pallas kernel here")


# speedup vs baseline: 219.0508x; 219.0508x over previous
import functools

import jax
import jax.numpy as jnp
from jax.experimental import pallas as pl
from jax.experimental.pallas import tpu as pltpu

_B, _C, _Z, _H, _W = 2, 4, 64, 128, 128
_ZB = 8


def _body(pred_ref, gt_ref, w_ref, out_ref):
    x = pred_ref[...]
    g = gt_ref[...]
    cls = jax.lax.broadcasted_iota(jnp.int32, x.shape, 1)
    sel = g[:, None, :, :, :] == cls
    sp = jax.nn.softplus(x)
    pw = w_ref[...].reshape(1, _C, 1, 1, 1)
    loss = sp + jnp.where(sel, pw * (sp - x) - sp, 0.0)
    part = jnp.sum(loss) * (1.0 / (_B * _C * _Z * _H * _W))

    @pl.when((pl.program_id(0) == 0) & (pl.program_id(1) == 0))
    def _init():
        out_ref[...] = jnp.zeros_like(out_ref)

    out_ref[...] += part


def kernel(pred, gt, weights):
    grid = (_B, _Z // _ZB)
    out = pl.pallas_call(
        _body,
        grid=grid,
        in_specs=[
            pl.BlockSpec((1, _C, _ZB, _H, _W), lambda b, z: (b, 0, z, 0, 0)),
            pl.BlockSpec((1, _ZB, _H, _W), lambda b, z: (b, z, 0, 0)),
            pl.BlockSpec((1, _C), lambda b, z: (0, 0)),
        ],
        out_specs=pl.BlockSpec((1, 1), lambda b, z: (0, 0)),
        out_shape=jax.ShapeDtypeStruct((1, 1), jnp.float32),
    )(pred, gt, weights.reshape(1, _C))
    return out[0, 0]
